# Initial kernel scaffold; baseline (speedup 1.0000x reference)
import jax
import jax.numpy as jnp
from jax.experimental import pallas as pl


def kernel(x, edge_index, edge_weight, W, b, gamma, beta):
    raise NotImplementedError("write your pallas kernel here")


# speedup vs baseline: 4.8202x; 4.8202x over previous
import functools

import jax
import jax.numpy as jnp
from jax import lax
from jax.experimental import pallas as pl
from jax.experimental.pallas import tpu as pltpu
from jax.experimental.pallas import tpu_sc as plsc

N = 49152
E = 442368
C_IN = 128
C_OUT = 128
G = 16
EPS = 1e-5

NCH = 4
CW = C_IN // NCH
NC = 2
NS = 16
EPT = E // NS
BE = 512
NBLK = EPT // BE
RPT = N // NS

BN = 2048
NBN = N // BN


def _make_lap(src_mul, chunk_mul):
    mesh = plsc.VectorSubcoreMesh(core_axis_name="c", subcore_axis_name="s")

    @functools.partial(
        pl.kernel,
        out_type=jax.ShapeDtypeStruct((NCH * N, CW), jnp.float32),
        mesh=mesh,
        compiler_params=pltpu.CompilerParams(use_tc_tiling_on_sc=False),
        scratch_types=[
            pltpu.VMEM((BE,), jnp.int32),
            pltpu.VMEM((BE,), jnp.int32),
            pltpu.VMEM((BE // 128, 128), jnp.int32),
            pltpu.VMEM((BE,), jnp.float32),
            pltpu.VMEM((BE, CW), jnp.float32),
            pltpu.VMEM_SHARED((N, CW), jnp.float32),
            pltpu.SemaphoreType.DMA,
        ],
    )
    def lap(table, src, dst2, w, zeros, out,
            src_v, idx_v, dst_v, w_v, rows_v, acc, sem):
        c = lax.axis_index("c")
        s = lax.axis_index("s")
        for cc in range(NCH // NC):
            chunk = c * (NCH // NC) + cc
            off = chunk * chunk_mul
            pltpu.sync_copy(zeros, acc.at[pl.ds(s * RPT, RPT)])
            plsc.subcore_barrier()

            def blk(b, carry):
                base = s * EPT + b * BE
                pltpu.sync_copy(src.at[pl.ds(base, BE)], src_v)
                pltpu.sync_copy(dst2.at[pl.ds(s * (EPT // 128) + b * (BE // 128), BE // 128)], dst_v)
                pltpu.sync_copy(w.at[pl.ds(base, BE)], w_v)

                def idxg(g, carry2):
                    sl = pl.ds(g * 16, 16)
                    idx_v[sl] = src_v[sl] * src_mul + off
                    return carry2
                lax.fori_loop(0, BE // 16, idxg, 0)

                pltpu.async_copy(table.at[idx_v], rows_v, sem).wait()

                def sgrp(g, carry2):
                    w16 = w_v[pl.ds(g * 16, 16)]
                    for t in range(16):
                        e = g * 16 + t
                        wt = w16[t]
                        rows_v[e, pl.ds(0, 16)] = rows_v[e, pl.ds(0, 16)] * wt
                        rows_v[e, pl.ds(16, 16)] = rows_v[e, pl.ds(16, 16)] * wt
                    return carry2
                lax.fori_loop(0, BE // 16, sgrp, 0)

                for j in range(BE // 128):
                    pltpu.sync_copy(rows_v.at[pl.ds(j * 128, 128)],
                                    acc.at[dst_v.at[j]], add=True)
                return carry
            lax.fori_loop(0, NBLK, blk, 0)
            plsc.subcore_barrier()
            pltpu.sync_copy(acc.at[pl.ds(s * RPT, RPT)],
                            out.at[pl.ds(chunk * N + s * RPT, RPT)])
            plsc.subcore_barrier()

    return lap


_lap1 = _make_lap(NCH, 1)
_lap2 = _make_lap(1, N)


def _mm_body(x_ref, y1_ref, z_ref, wx_ref, w1_ref, w2_ref, b_ref,
             y_ref, sum_ref, ssq_ref):
    i = pl.program_id(0)
    acc = jnp.dot(x_ref[...], wx_ref[...], preferred_element_type=jnp.float32)
    for cidx in range(NCH):
        acc += jnp.dot(y1_ref[cidx], w1_ref[cidx],
                       preferred_element_type=jnp.float32)
        acc += jnp.dot(z_ref[cidx], w2_ref[cidx],
                       preferred_element_type=jnp.float32)
    acc += b_ref[...]
    y_ref[...] = acc

    @pl.when(i == 0)
    def _():
        sum_ref[...] = jnp.zeros_like(sum_ref)
        ssq_ref[...] = jnp.zeros_like(ssq_ref)

    sum_ref[...] += jnp.sum(acc, axis=0, keepdims=True)
    ssq_ref[...] += jnp.sum(acc * acc, axis=0, keepdims=True)


def _gn_body(y_ref, sum_ref, ssq_ref, gamma_ref, beta_ref, o_ref):
    cnt = jnp.float32(N * (C_OUT // G))
    ii = lax.broadcasted_iota(jnp.int32, (C_OUT, C_OUT), 0) // (C_OUT // G)
    jj = lax.broadcasted_iota(jnp.int32, (C_OUT, C_OUT), 1) // (C_OUT // G)
    m = (ii == jj).astype(jnp.float32)
    gs = jnp.dot(sum_ref[...], m, preferred_element_type=jnp.float32)
    gq = jnp.dot(ssq_ref[...], m, preferred_element_type=jnp.float32)
    mean = gs / cnt
    var = gq / cnt - mean * mean
    rstd = lax.rsqrt(var + EPS)
    scale = rstd * gamma_ref[...]
    shift = beta_ref[...] - mean * scale
    o_ref[...] = jnp.maximum(y_ref[...] * scale + shift, 0.0)


def kernel(x, edge_index, edge_weight, W, b, gamma, beta):
    src = edge_index[0]
    dst2 = edge_index[1].reshape(E // 128, 128)
    zeros = jnp.zeros((RPT, CW), jnp.float32)

    x2d = x.reshape(N * NCH, CW)
    y1ch = _lap1(x2d, src, dst2, edge_weight, zeros)
    zch = _lap2(y1ch, src, dst2, edge_weight, zeros)

    w0, w1, w2 = W[:C_IN], W[C_IN:2 * C_IN], W[2 * C_IN:]
    wx = w0 - w2
    w1c = w1.reshape(NCH, CW, C_OUT)
    w2c = (2.0 * w2).reshape(NCH, CW, C_OUT)

    y_raw, ssum, ssq = pl.pallas_call(
        _mm_body,
        grid=(NBN,),
        in_specs=[
            pl.BlockSpec((BN, C_IN), lambda i: (i, 0)),
            pl.BlockSpec((NCH, BN, CW), lambda i: (0, i, 0)),
            pl.BlockSpec((NCH, BN, CW), lambda i: (0, i, 0)),
            pl.BlockSpec((C_IN, C_OUT), lambda i: (0, 0)),
            pl.BlockSpec((NCH, CW, C_OUT), lambda i: (0, 0, 0)),
            pl.BlockSpec((NCH, CW, C_OUT), lambda i: (0, 0, 0)),
            pl.BlockSpec((1, C_OUT), lambda i: (0, 0)),
        ],
        out_specs=[
            pl.BlockSpec((BN, C_OUT), lambda i: (i, 0)),
            pl.BlockSpec((1, C_OUT), lambda i: (0, 0)),
            pl.BlockSpec((1, C_OUT), lambda i: (0, 0)),
        ],
        out_shape=[
            jax.ShapeDtypeStruct((N, C_OUT), jnp.float32),
            jax.ShapeDtypeStruct((1, C_OUT), jnp.float32),
            jax.ShapeDtypeStruct((1, C_OUT), jnp.float32),
        ],
    )(x.reshape(N, C_IN), y1ch.reshape(NCH, N, CW), zch.reshape(NCH, N, CW),
      wx, w1c, w2c, b.reshape(1, C_OUT))

    out = pl.pallas_call(
        _gn_body,
        grid=(NBN,),
        in_specs=[
            pl.BlockSpec((BN, C_OUT), lambda i: (i, 0)),
            pl.BlockSpec((1, C_OUT), lambda i: (0, 0)),
            pl.BlockSpec((1, C_OUT), lambda i: (0, 0)),
            pl.BlockSpec((1, C_OUT), lambda i: (0, 0)),
            pl.BlockSpec((1, C_OUT), lambda i: (0, 0)),
        ],
        out_specs=pl.BlockSpec((BN, C_OUT), lambda i: (i, 0)),
        out_shape=jax.ShapeDtypeStruct((N, C_OUT), jnp.float32),
    )(y_raw, ssum, ssq, gamma.reshape(1, C_OUT), beta.reshape(1, C_OUT))

    return out.reshape(1, N, C_OUT)
